# Initial kernel scaffold; baseline (speedup 1.0000x reference)
import jax
import jax.numpy as jnp
from jax.experimental import pallas as pl


def kernel(input, grid, fScaleWidth, fScaleHeight):
    raise NotImplementedError("write your pallas kernel here")


# speedup vs baseline: 1.3399x; 1.3399x over previous
import functools

import jax
import jax.numpy as jnp
from jax import lax
from jax.experimental import pallas as pl
from jax.experimental.pallas import tpu as pltpu
from jax.experimental.pallas import tpu_sc as plsc

B, C, H, W, Q = 4, 192, 96, 96, 8192
HW = H * W
NCORE, NSUB = 2, 16
CHG = C // NSUB
BPG = B // NCORE
QC = 256
NG = QC // 16
NCHUNK = Q // QC

@functools.lru_cache(maxsize=1)
def _build():
    mesh = plsc.VectorSubcoreMesh(
        core_axis_name="c", subcore_axis_name="s",
        num_cores=NCORE, num_subcores=NSUB)
    return functools.partial(
        pl.kernel,
        out_type=jax.ShapeDtypeStruct((B, C, 4, Q), jnp.float32),
        mesh=mesh,
        compiler_params=pltpu.CompilerParams(needs_layout_passes=False),
        scratch_types=[
            pltpu.VMEM((CHG * HW,), jnp.float32),
            pltpu.VMEM((CHG, 4, QC), jnp.float32),
            pltpu.VMEM((QC,), jnp.float32),
            pltpu.VMEM((QC,), jnp.float32),
            pltpu.VMEM((QC,), jnp.float32),
            pltpu.VMEM((16,), jnp.float32),
            pltpu.VMEM((16,), jnp.float32),
        ],
    )(_dgs_sc)


def _dgs_sc(feat_hbm, gx_hbm, gy_hbm, gz_hbm, fsw_hbm, fsh_hbm, out_hbm,
            feat_v, stage_v, xv, yv, zv, fswv, fshv):
    cid = lax.axis_index("c")
    sid = lax.axis_index("s")
    cg = sid
    bp = cid

    for bi in range(BPG):
        b = bp * BPG + bi
        pltpu.sync_copy(feat_hbm.at[pl.ds((b * C + cg * CHG) * HW, CHG * HW)],
                        feat_v)
        pltpu.sync_copy(fsw_hbm.at[b], fswv)
        pltpu.sync_copy(fsh_hbm.at[b], fshv)
        fw = fswv[...]
        fh = fshv[...]

        def chunk_body(ch, _, b=b, fw=fw, fh=fh):
            q0 = ch * QC
            pltpu.sync_copy(gx_hbm.at[b, pl.ds(q0, QC)], xv)
            pltpu.sync_copy(gy_hbm.at[b, pl.ds(q0, QC)], yv)
            pltpu.sync_copy(gz_hbm.at[b, pl.ds(q0, QC)], zv)

            def group_body(g, _):
                off = g * 16
                xq = xv[pl.ds(off, 16)]
                yq = yv[pl.ds(off, 16)]
                zq = zv[pl.ds(off, 16)]
                ix = jnp.clip((xq + 1.0) * (0.5 * (W - 1)), 0.0, W - 1.0)
                iy = jnp.clip((yq + 1.0) * (0.5 * (H - 1)), 0.0, H - 1.0)
                x0i = jnp.minimum(ix.astype(jnp.int32), W - 2)
                y0i = jnp.minimum(iy.astype(jnp.int32), H - 2)
                dx = ix - x0i.astype(jnp.float32)
                dy = iy - y0i.astype(jnp.float32)
                pix = y0i * W + x0i
                omx = 1.0 - dx
                omy = 1.0 - dy
                rz = 1.0 / zq
                sw = fw * rz
                sh = fh * rz
                sx = xq * rz
                sy = yq * rz
                for c in range(CHG):
                    base = pix + (c * HW)
                    f00 = plsc.load_gather(feat_v, [base])
                    f01 = plsc.load_gather(feat_v, [base + 1])
                    f10 = plsc.load_gather(feat_v, [base + W])
                    f11 = plsc.load_gather(feat_v, [base + (W + 1)])
                    g0 = f01 - f00
                    g1 = f11 - f10
                    phi = omy * (f00 + dx * g0) + dy * (f10 + dx * g1)
                    dj = omy * g0 + dy * g1
                    di = omx * (f10 - f00) + dx * (f11 - f01)
                    stage_v[c, 0, pl.ds(off, 16)] = phi
                    stage_v[c, 1, pl.ds(off, 16)] = dj * sw
                    stage_v[c, 2, pl.ds(off, 16)] = di * sh
                    stage_v[c, 3, pl.ds(off, 16)] = -(di * sy) - dj * sx
                return 0

            lax.fori_loop(0, NG, group_body, 0)
            pltpu.sync_copy(stage_v,
                            out_hbm.at[b, pl.ds(cg * CHG, CHG), :,
                                       pl.ds(q0, QC)])
            return 0

        lax.fori_loop(0, NCHUNK, chunk_body, 0)


def kernel(input, grid, fScaleWidth, fScaleHeight):
    feat = input.reshape(B * C * HW)
    gx = grid[:, :, 0]
    gy = grid[:, :, 1]
    gz = grid[:, :, 2]
    fsw = jnp.broadcast_to(fScaleWidth[:, None], (B, 16))
    fsh = jnp.broadcast_to(fScaleHeight[:, None], (B, 16))
    return _build()(feat, gx, gy, gz, fsw, fsh)
